# Initial kernel scaffold; baseline (speedup 1.0000x reference)
import jax
import jax.numpy as jnp
from jax.experimental import pallas as pl


def kernel(edge_index, x, W1, b1, W2, b2):
    raise NotImplementedError("write your pallas kernel here")


# speedup vs baseline: 33.2315x; 33.2315x over previous
import functools

import jax
import jax.numpy as jnp
from jax import lax
from jax.experimental import pallas as pl
from jax.experimental.pallas import tpu as pltpu
from jax.experimental.pallas import tpu_sc as plsc

_NC = 2
_NS = 16
_NW = _NC * _NS
_K = 125


def _sc_mesh():
    return plsc.VectorSubcoreMesh(
        core_axis_name="c", subcore_axis_name="s",
        num_cores=_NC, num_subcores=_NS)


def _degree_sc(dst3, n):
    nchunks = dst3.shape[1]
    half = n // 2
    hbuf = (half + 15) // 16 * 16
    obase = hbuf
    obuf = (_K + 15) // 16 * 16

    @functools.partial(
        pl.kernel,
        out_type=(jax.ShapeDtypeStruct((n,), jnp.float32),
                  jax.ShapeDtypeStruct((n,), jnp.float32)),
        mesh=_sc_mesh(),
        scratch_types=[
            pltpu.VMEM((nchunks, _K), jnp.int32),
            pltpu.VMEM((hbuf + obuf,), jnp.float32),
            pltpu.VMEM_SHARED((n,), jnp.float32),
        ],
    )
    def run(dst_hbm, deg0_hbm, deg1_hbm, idx_v, ones_v, deg_sh):
        c = lax.axis_index("c")
        s = lax.axis_index("s")
        w = s * _NC + c

        def fill(i, carry):
            ones_v[pl.ds(pl.multiple_of(i * 16, 16), 16)] = jnp.full(
                (16,), 0.5, jnp.float32)
            return carry
        lax.fori_loop(0, hbuf // 16, fill, 0)

        def fill1(i, carry):
            ones_v[pl.ds(pl.multiple_of(obase + i * 16, 16), 16)] = jnp.ones(
                (16,), jnp.float32)
            return carry
        lax.fori_loop(0, obuf // 16, fill1, 0)

        @pl.when(s == 0)
        def _():
            pltpu.sync_copy(ones_v.at[pl.ds(0, half)], deg_sh.at[pl.ds(0, half)])

        @pl.when(s == 1)
        def _():
            pltpu.sync_copy(ones_v.at[pl.ds(0, half)],
                            deg_sh.at[pl.ds(half, half)])

        pltpu.sync_copy(dst_hbm.at[w], idx_v)
        plsc.subcore_barrier()

        ones_k = ones_v.at[pl.ds(obase, _K)]

        def step(j, carry):
            pltpu.sync_copy(ones_k, deg_sh.at[idx_v.at[j]], add=True)
            return carry
        lax.fori_loop(0, nchunks, step, 0)

        plsc.subcore_barrier()

        @pl.when(jnp.logical_and(s == 0, c == 0))
        def _():
            pltpu.sync_copy(deg_sh, deg0_hbm)

        @pl.when(jnp.logical_and(s == 0, c == 1))
        def _():
            pltpu.sync_copy(deg_sh, deg1_hbm)

    return run(dst3)


def _segsum_sc(src3, dst3, y):
    nchunks = src3.shape[1]
    n, d = y.shape
    rpt = -(-n // (_NS * 128)) * 128
    n_pad = rpt * _NS

    hchunks = nchunks // 2

    @functools.partial(
        pl.kernel,
        out_type=jax.ShapeDtypeStruct((_NC, n_pad, d), jnp.float32),
        mesh=_sc_mesh(),
        scratch_types=[
            pltpu.VMEM((hchunks, _K), jnp.int32),
            pltpu.VMEM((hchunks, _K), jnp.int32),
            pltpu.VMEM((2, 128, d), jnp.float32),
            pltpu.VMEM_SHARED((n_pad, d), jnp.float32),
            pltpu.SemaphoreType.DMA,
            pltpu.SemaphoreType.DMA,
        ],
    )
    def run(src_hbm, dst_hbm, y_hbm, out_hbm, si_v, di_v, rows_v, acc_sh,
            sem0, sem1):
        c = lax.axis_index("c")
        s = lax.axis_index("s")
        w = s * _NC + c

        def zrow(i, carry):
            for jj in range(8):
                rows_v[0, i, pl.ds(jj * 16, 16)] = jnp.zeros((16,), jnp.float32)
            return carry
        lax.fori_loop(0, 128, zrow, 0)
        for k in range(rpt // 128):
            pltpu.sync_copy(rows_v.at[0],
                            acc_sh.at[pl.ds(s * rpt + k * 128, 128)])
        plsc.subcore_barrier()

        gb = (rows_v.at[0].at[pl.ds(0, _K)], rows_v.at[1].at[pl.ds(0, _K)])
        sems = (sem0, sem1)
        for h in range(2):
            pltpu.sync_copy(src_hbm.at[w, pl.ds(h * hchunks, hchunks)], si_v)
            pltpu.sync_copy(dst_hbm.at[w, pl.ds(h * hchunks, hchunks)], di_v)

            pltpu.async_copy(y_hbm.at[si_v.at[0]], gb[0], sems[0])
            pltpu.async_copy(y_hbm.at[si_v.at[1]], gb[1], sems[1])

            def step(i, carry):
                for b in range(2):
                    j = 2 * i + b
                    pltpu.make_async_copy(
                        y_hbm.at[si_v.at[j]], gb[b], sems[b]).wait()
                    pltpu.sync_copy(gb[b], acc_sh.at[di_v.at[j]], add=True)

                    @pl.when(j + 2 < hchunks)
                    def _():
                        pltpu.async_copy(
                            y_hbm.at[si_v.at[j + 2]], gb[b], sems[b])
                return carry
            lax.fori_loop(0, hchunks // 2, step, 0)

        plsc.subcore_barrier()
        pltpu.sync_copy(acc_sh.at[pl.ds(s * rpt, rpt)],
                        out_hbm.at[c, pl.ds(s * rpt, rpt)])

    return run(src3, dst3, y)


_BR = 1000


def _mm_scale_tc(x, w, deg_t):
    n, d = x.shape

    def body(x_ref, w_ref, deg_ref, o_ref):
        dinv = lax.rsqrt(deg_ref[:, :1] + deg_ref[:, 1:2])
        y = jnp.dot(x_ref[...], w_ref[...], preferred_element_type=jnp.float32)
        o_ref[...] = y * dinv

    return pl.pallas_call(
        body,
        grid=(n // _BR,),
        in_specs=[
            pl.BlockSpec((_BR, d), lambda i: (i, 0)),
            pl.BlockSpec((d, d), lambda i: (0, 0)),
            pl.BlockSpec((_BR, 2), lambda i: (i, 0)),
        ],
        out_specs=pl.BlockSpec((_BR, d), lambda i: (i, 0)),
        out_shape=jax.ShapeDtypeStruct((n, d), jnp.float32),
    )(x, w, deg_t)


def _mid_layer_tc(acc, y1, deg_t, b, w):
    n, d = y1.shape

    def body(acc_ref, y1_ref, deg_ref, b_ref, w_ref, o_ref):
        dinv = lax.rsqrt(deg_ref[:, :1] + deg_ref[:, 1:2])
        tot = acc_ref[0] + acc_ref[1] + y1_ref[...]
        h = jnp.maximum(tot * dinv + b_ref[...], 0.0)
        y2 = jnp.dot(h, w_ref[...], preferred_element_type=jnp.float32)
        o_ref[...] = y2 * dinv

    return pl.pallas_call(
        body,
        grid=(n // _BR,),
        in_specs=[
            pl.BlockSpec((_NC, _BR, d), lambda i: (0, i, 0)),
            pl.BlockSpec((_BR, d), lambda i: (i, 0)),
            pl.BlockSpec((_BR, 2), lambda i: (i, 0)),
            pl.BlockSpec((1, d), lambda i: (0, 0)),
            pl.BlockSpec((d, d), lambda i: (0, 0)),
        ],
        out_specs=pl.BlockSpec((_BR, d), lambda i: (i, 0)),
        out_shape=jax.ShapeDtypeStruct((n, d), jnp.float32),
    )(acc, y1, deg_t, b, w)


def _final_tc(acc, y2, deg_t, b):
    n, d = y2.shape

    def body(acc_ref, y2_ref, deg_ref, b_ref, o_ref):
        dinv = lax.rsqrt(deg_ref[:, :1] + deg_ref[:, 1:2])
        tot = acc_ref[0] + acc_ref[1] + y2_ref[...]
        o_ref[...] = jnp.maximum(tot * dinv + b_ref[...], 0.0)

    return pl.pallas_call(
        body,
        grid=(n // _BR,),
        in_specs=[
            pl.BlockSpec((_NC, _BR, d), lambda i: (0, i, 0)),
            pl.BlockSpec((_BR, d), lambda i: (i, 0)),
            pl.BlockSpec((_BR, 2), lambda i: (i, 0)),
            pl.BlockSpec((1, d), lambda i: (0, 0)),
        ],
        out_specs=pl.BlockSpec((_BR, d), lambda i: (i, 0)),
        out_shape=jax.ShapeDtypeStruct((n, d), jnp.float32),
    )(acc, y2, deg_t, b)


def kernel(edge_index, x, W1, b1, W2, b2):
    n, d = x.shape
    e = edge_index.shape[1]
    assert e % (_NW * _K) == 0 and n % _NS == 0 and (n // 2) % 8 == 0
    assert n % _BR == 0

    src3 = edge_index[0].reshape(_NW, e // (_NW * _K), _K)
    dst3 = edge_index[1].reshape(_NW, e // (_NW * _K), _K)

    deg0, deg1 = _degree_sc(dst3, n)
    deg_t = jnp.stack([deg0, deg1], axis=1)
    b1r = b1.reshape(1, d)
    b2r = b2.reshape(1, d)

    y1 = _mm_scale_tc(x, W1, deg_t)
    acc1 = _segsum_sc(src3, dst3, y1)
    y2 = _mid_layer_tc(acc1, y1, deg_t, b1r, W2)
    acc2 = _segsum_sc(src3, dst3, y2)
    return _final_tc(acc2, y2, deg_t, b2r)
